# Initial kernel scaffold; baseline (speedup 1.0000x reference)
import jax
import jax.numpy as jnp
from jax.experimental import pallas as pl


def kernel(input_seq, input_mask, table):
    raise NotImplementedError("write your pallas kernel here")


# speedup vs baseline: 1.0673x; 1.0673x over previous
import jax
import jax.numpy as jnp
from jax import lax
from jax.experimental import pallas as pl
from jax.experimental.pallas import tpu as pltpu
from jax.experimental.pallas import tpu_sc as plsc

NC, NS, LANES = 2, 16, 16
NW = NC * NS
NBUF = 4


def _encoder_avg_body(seq_hbm, mask_hbm, table_hbm, out_hbm,
                      seq_v, tgt_v, gb0, gb1, gb2, gb3, acc_v, cnt_v, shacc,
                      gs0, gs1, gs2, gs3, ss0, ss1, ss2, ss3):
    L, BPW = seq_v.shape
    D = acc_v.shape[1]
    KD = D // LANES
    KB = BPW // LANES
    gb = (gb0, gb1, gb2, gb3)
    gsem = (gs0, gs1, gs2, gs3)
    ssem = (ss0, ss1, ss2, ss3)

    sid = lax.axis_index("s")
    wid = sid * NC + lax.axis_index("c")
    base = wid * BPW
    srow = sid * (BPW + 1)

    pltpu.sync_copy(seq_hbm.at[:, pl.ds(base, BPW)], seq_v)
    pltpu.sync_copy(mask_hbm.at[:, pl.ds(base, BPW)], tgt_v)

    zero = jnp.zeros((LANES,), jnp.float32)

    def zbody(i, c):
        for k in range(KD):
            acc_v[i, pl.ds(k * LANES, LANES)] = zero
        return c

    lax.fori_loop(0, BPW + 1, zbody, 0)
    pltpu.sync_copy(acc_v, shacc.at[pl.ds(srow, BPW + 1)])

    iotas = [jnp.arange(k * LANES, (k + 1) * LANES, dtype=jnp.int32)
             for k in range(KB)]
    trash = jnp.full((LANES,), BPW, jnp.int32)

    def cbody(l, cnts):
        out = []
        for k in range(KB):
            m = tgt_v[l, pl.ds(k * LANES, LANES)]
            tgt_v[l, pl.ds(k * LANES, LANES)] = srow + jnp.where(
                m != 0, iotas[k], trash)
            out.append(cnts[k] + m)
        return tuple(out)

    cnts = lax.fori_loop(
        0, L, cbody,
        tuple(jnp.zeros((LANES,), jnp.int32) for _ in range(KB)))
    for k in range(KB):
        cnt_v[pl.ds(k * LANES, LANES)] = 1.0 / cnts[k].astype(jnp.float32)

    for b in range(NBUF):
        pltpu.async_copy(table_hbm.at[seq_v.at[b]], gb[b], gsem[b])

    def step(l, b, issue_next):
        pltpu.make_async_copy(table_hbm.at[seq_v.at[l]], gb[b], gsem[b]).wait()
        pltpu.async_copy(gb[b], shacc.at[tgt_v.at[l]], ssem[b], add=True)
        pltpu.make_async_copy(gb[b], shacc.at[tgt_v.at[l]], ssem[b]).wait()
        if issue_next:
            pltpu.async_copy(table_hbm.at[seq_v.at[l + NBUF]], gb[b], gsem[b])

    NG = L // NBUF

    def gbody(g, c):
        for b in range(NBUF):
            step(g * NBUF + b, b, True)
        return c

    lax.fori_loop(0, NG - 1, gbody, 0)
    for b in range(NBUF):
        step((NG - 1) * NBUF + b, b, False)

    pltpu.sync_copy(shacc.at[pl.ds(srow, BPW)], acc_v.at[pl.ds(0, BPW)])

    lane_iota = jnp.arange(LANES, dtype=jnp.int32)

    def dbody(i, c):
        grp = i // LANES
        lane = i - grp * LANES
        rv = cnt_v[pl.ds(grp * LANES, LANES)]
        w = jnp.sum(jnp.where(lane_iota == lane, rv, 0.0))
        wv = jnp.full((LANES,), w, jnp.float32)
        for k in range(KD):
            sl = pl.ds(k * LANES, LANES)
            acc_v[i, sl] = acc_v[i, sl] * wv
        return c

    lax.fori_loop(0, BPW, dbody, 0)

    pltpu.sync_copy(acc_v.at[pl.ds(0, BPW)], out_hbm.at[pl.ds(base, BPW)])


def kernel(input_seq, input_mask, table):
    L, B = input_seq.shape
    V, D = table.shape
    BPW = B // NW
    mesh = plsc.VectorSubcoreMesh(core_axis_name="c", subcore_axis_name="s",
                                  num_cores=NC, num_subcores=NS)
    run = pl.kernel(
        _encoder_avg_body,
        out_type=jax.ShapeDtypeStruct((B, D), jnp.float32),
        mesh=mesh,
        compiler_params=pltpu.CompilerParams(needs_layout_passes=False,
                                             use_tc_tiling_on_sc=False),
        scratch_types=[
            pltpu.VMEM((L, BPW), jnp.int32),
            pltpu.VMEM((L, BPW), jnp.int32),
            *[pltpu.VMEM((BPW, D), jnp.float32) for _ in range(NBUF)],
            pltpu.VMEM((BPW + 1, D), jnp.float32),
            pltpu.VMEM((BPW,), jnp.float32),
            pltpu.VMEM_SHARED((NS * (BPW + 1), D), jnp.float32),
            *[pltpu.SemaphoreType.DMA for _ in range(2 * NBUF)],
        ],
    )
    return run(input_seq, input_mask, table)
